# baseline (speedup 1.0000x reference)
import functools

import jax
import jax.numpy as jnp
from jax import lax
from jax.experimental import pallas as pl
from jax.experimental.pallas import tpu as pltpu
from jax.experimental.pallas import tpu_sc as plsc

_SUB = 32
_NBUF = 2


def kernel(inputs, pos_embedding):
    B, S = inputs.shape
    P, D = pos_embedding.shape

    info = plsc.get_sparse_core_info()
    NC, NS = info.num_cores, info.num_subcores
    NW = NC * NS
    RPW = S // NW
    NSUB = RPW // _SUB

    mesh = plsc.VectorSubcoreMesh(core_axis_name="c", subcore_axis_name="s")

    @functools.partial(
        pl.kernel,
        mesh=mesh,
        out_type=jax.ShapeDtypeStruct((B, S, D), pos_embedding.dtype),
        scratch_types=[
            pltpu.VMEM((_NBUF, _SUB, D), pos_embedding.dtype),
            pltpu.VMEM_SHARED((NS, _NBUF, _SUB, D), pos_embedding.dtype),
            pltpu.SemaphoreType.DMA((_NBUF,)),
            pltpu.SemaphoreType.DMA((_NBUF,)),
            pltpu.SemaphoreType.DMA((_NBUF,)),
            pltpu.SemaphoreType.DMA((_NBUF,)),
        ],
    )
    def sc_copy(table_hbm, out_hbm, buf, shbuf, insem, outsem, shinsem, shoutsem):
        sid = lax.axis_index("s")
        wid = sid * NC + lax.axis_index("c")
        base = wid * RPW

        def in_copy(j):
            return pltpu.make_async_copy(
                table_hbm.at[pl.ds(base + j * _SUB, _SUB), :],
                buf.at[j % _NBUF],
                insem.at[j % _NBUF],
            )

        def sh_in_copy(j):
            return pltpu.make_async_copy(
                table_hbm.at[pl.ds(base + j * _SUB, _SUB), :],
                shbuf.at[sid, j % _NBUF],
                shinsem.at[j % _NBUF],
            )

        def out_copies(j):
            cs = [
                pltpu.make_async_copy(
                    buf.at[j % _NBUF],
                    out_hbm.at[b, pl.ds(base + j * _SUB, _SUB), :],
                    outsem.at[j % _NBUF],
                )
                for b in range(B - 1)
            ]
            cs.append(
                pltpu.make_async_copy(
                    shbuf.at[sid, j % _NBUF],
                    out_hbm.at[B - 1, pl.ds(base + j * _SUB, _SUB), :],
                    shoutsem.at[j % _NBUF],
                )
            )
            return cs

        pending = {}
        in_copy(0).start()
        sh_in_copy(0).start()
        for j in range(NSUB):
            nxt = j + 1
            if nxt < NSUB:
                prev = nxt - _NBUF
                if prev >= 0:
                    for c in pending.pop(prev):
                        c.wait()
                in_copy(nxt).start()
                sh_in_copy(nxt).start()
            in_copy(j).wait()
            sh_in_copy(j).wait()
            cs = out_copies(j)
            for c in cs:
                c.start()
            pending[j] = cs
        for j in sorted(pending):
            for c in pending[j]:
                c.wait()

    return sc_copy(pos_embedding)


# speedup vs baseline: 1.0926x; 1.0926x over previous
import functools

import jax
import jax.numpy as jnp
from jax import lax
from jax.experimental import pallas as pl
from jax.experimental.pallas import tpu as pltpu
from jax.experimental.pallas import tpu_sc as plsc

_SUB = 16
_NBUF = 4


def kernel(inputs, pos_embedding):
    B, S = inputs.shape
    P, D = pos_embedding.shape

    info = plsc.get_sparse_core_info()
    NC, NS = info.num_cores, info.num_subcores
    NW = NC * NS
    RPW = S // NW
    NSUB = RPW // _SUB

    mesh = plsc.VectorSubcoreMesh(core_axis_name="c", subcore_axis_name="s")

    @functools.partial(
        pl.kernel,
        mesh=mesh,
        out_type=jax.ShapeDtypeStruct((B, S, D), pos_embedding.dtype),
        scratch_types=[
            pltpu.VMEM((_NBUF, _SUB, D), pos_embedding.dtype),
            pltpu.SemaphoreType.DMA((_NBUF,)),
            pltpu.SemaphoreType.DMA((_NBUF,)),
        ],
    )
    def sc_copy(table_hbm, out_hbm, buf, insem, outsem):
        wid = lax.axis_index("s") * NC + lax.axis_index("c")
        base = wid * RPW

        def in_copy(j):
            return pltpu.make_async_copy(
                table_hbm.at[pl.ds(base + j * _SUB, _SUB), :],
                buf.at[j % _NBUF],
                insem.at[j % _NBUF],
            )

        def out_copies(j):
            return [
                pltpu.make_async_copy(
                    buf.at[j % _NBUF],
                    out_hbm.at[b, pl.ds(base + j * _SUB, _SUB), :],
                    outsem.at[j % _NBUF],
                )
                for b in range(B)
            ]

        pending = {}
        in_copy(0).start()
        for j in range(NSUB):
            nxt = j + 1
            if nxt < NSUB:
                prev = nxt - _NBUF
                if prev >= 0:
                    for c in pending.pop(prev):
                        c.wait()
                in_copy(nxt).start()
            in_copy(j).wait()
            cs = out_copies(j)
            for c in cs:
                c.start()
            pending[j] = cs
        for j in sorted(pending):
            for c in pending[j]:
                c.wait()

    return sc_copy(pos_embedding)


# speedup vs baseline: 1.1531x; 1.0553x over previous
import functools

import jax
import jax.numpy as jnp
from jax import lax
from jax.experimental import pallas as pl
from jax.experimental.pallas import tpu as pltpu
from jax.experimental.pallas import tpu_sc as plsc

_SUB = 32
_NBUF = 3


def kernel(inputs, pos_embedding):
    B, S = inputs.shape
    P, D = pos_embedding.shape

    info = plsc.get_sparse_core_info()
    NC, NS = info.num_cores, info.num_subcores
    NW = NC * NS
    RPW = S // NW
    NSUB = RPW // _SUB

    mesh = plsc.VectorSubcoreMesh(core_axis_name="c", subcore_axis_name="s")

    @functools.partial(
        pl.kernel,
        mesh=mesh,
        out_type=jax.ShapeDtypeStruct((B, S, D), pos_embedding.dtype),
        scratch_types=[
            pltpu.VMEM((_NBUF, _SUB, D), pos_embedding.dtype),
            pltpu.SemaphoreType.DMA((_NBUF,)),
            pltpu.SemaphoreType.DMA((_NBUF,)),
        ],
    )
    def sc_copy(table_hbm, out_hbm, buf, insem, outsem):
        wid = lax.axis_index("s") * NC + lax.axis_index("c")
        base = wid * RPW

        def in_copy(j):
            return pltpu.make_async_copy(
                table_hbm.at[pl.ds(base + j * _SUB, _SUB), :],
                buf.at[j % _NBUF],
                insem.at[j % _NBUF],
            )

        def out_copies(j):
            return [
                pltpu.make_async_copy(
                    buf.at[j % _NBUF],
                    out_hbm.at[b, pl.ds(base + j * _SUB, _SUB), :],
                    outsem.at[j % _NBUF],
                )
                for b in range(B)
            ]

        pending = {}
        in_copy(0).start()
        for j in range(NSUB):
            nxt = j + 1
            if nxt < NSUB:
                prev = nxt - _NBUF
                if prev >= 0:
                    for c in pending.pop(prev):
                        c.wait()
                in_copy(nxt).start()
            in_copy(j).wait()
            cs = out_copies(j)
            for c in cs:
                c.start()
            pending[j] = cs
        for j in sorted(pending):
            for c in pending[j]:
                c.wait()

    return sc_copy(pos_embedding)
